# Initial kernel scaffold; baseline (speedup 1.0000x reference)
import jax
import jax.numpy as jnp
from jax.experimental import pallas as pl


def kernel(hidden_states, weight):
    raise NotImplementedError("write your pallas kernel here")


# speedup vs baseline: 1.0885x; 1.0885x over previous
import jax
import jax.numpy as jnp
from jax.experimental import pallas as pl

_NUM_EXPERTS = 64
_TOP_K = 8
_HIDDEN = 4096
_ROWS_PER_BLOCK = 512


def _router_kernel(hs_ref, w_ref, probs_ref, scores_ref, idx_ref):
    hs = hs_ref[...]
    w = w_ref[...]
    logits = jax.lax.dot_general(
        hs, w, (((1,), (1,)), ((), ())), preferred_element_type=jnp.float32
    )
    m = jnp.max(logits, axis=-1, keepdims=True)
    e = jnp.exp(logits - m)
    probs = e / jnp.sum(e, axis=-1, keepdims=True)
    probs_ref[...] = probs

    col = jax.lax.broadcasted_iota(jnp.int32, probs.shape, 1)
    cur = probs
    vals = []
    idxs = []
    for _ in range(_TOP_K):
        mv = jnp.max(cur, axis=-1, keepdims=True)
        im = jnp.min(
            jnp.where(cur == mv, col, _NUM_EXPERTS), axis=-1, keepdims=True
        )
        vals.append(mv)
        idxs.append(im)
        cur = jnp.where(col == im, -1.0, cur)
    v = jnp.concatenate(vals, axis=-1)
    i = jnp.concatenate(idxs, axis=-1)
    v = v / jnp.sum(v, axis=-1, keepdims=True)
    scores_ref[...] = v
    idx_ref[...] = i


def kernel(hidden_states, weight):
    hs = hidden_states.reshape(-1, _HIDDEN)
    n = hs.shape[0]
    grid = n // _ROWS_PER_BLOCK
    probs, scores, idx = pl.pallas_call(
        _router_kernel,
        grid=(grid,),
        in_specs=[
            pl.BlockSpec((_ROWS_PER_BLOCK, _HIDDEN), lambda i: (i, 0)),
            pl.BlockSpec((_NUM_EXPERTS, _HIDDEN), lambda i: (0, 0)),
        ],
        out_specs=[
            pl.BlockSpec((_ROWS_PER_BLOCK, _NUM_EXPERTS), lambda i: (i, 0)),
            pl.BlockSpec((_ROWS_PER_BLOCK, _TOP_K), lambda i: (i, 0)),
            pl.BlockSpec((_ROWS_PER_BLOCK, _TOP_K), lambda i: (i, 0)),
        ],
        out_shape=[
            jax.ShapeDtypeStruct((n, _NUM_EXPERTS), jnp.float32),
            jax.ShapeDtypeStruct((n, _TOP_K), jnp.float32),
            jax.ShapeDtypeStruct((n, _TOP_K), jnp.int32),
        ],
    )(hs, weight)
    return (probs, scores, idx)


# speedup vs baseline: 1.2999x; 1.1942x over previous
import jax
import jax.numpy as jnp
from jax.experimental import pallas as pl

_NUM_EXPERTS = 64
_TOP_K = 8
_HIDDEN = 4096
_ROWS_PER_BLOCK = 512


def _router_kernel(hs_ref, w_ref, probs_ref, scores_ref, idx_ref):
    hs = hs_ref[...]
    w = w_ref[...]
    logits = jax.lax.dot_general(
        hs, w, (((1,), (1,)), ((), ())), preferred_element_type=jnp.float32
    )
    m = jnp.max(logits, axis=-1, keepdims=True)
    e = jnp.exp(logits - m)
    probs = e / jnp.sum(e, axis=-1, keepdims=True)
    probs_ref[...] = probs

    col = jax.lax.broadcasted_iota(jnp.int32, probs.shape, 1)
    bits = jax.lax.bitcast_convert_type(probs, jnp.int32)
    key = jax.lax.bitcast_convert_type(
        (bits & jnp.int32(~63)) | (jnp.int32(63) - col), jnp.float32
    )
    vals = []
    idxs = []
    for _ in range(_TOP_K):
        kmax = jnp.max(key, axis=-1, keepdims=True)
        kbits = jax.lax.bitcast_convert_type(kmax, jnp.int32)
        vals.append(
            jax.lax.bitcast_convert_type(kbits & jnp.int32(~63), jnp.float32)
        )
        idxs.append(jnp.int32(63) - (kbits & jnp.int32(63)))
        key = jnp.where(key == kmax, jnp.float32(-1.0), key)
    v = jnp.concatenate(vals, axis=-1)
    i = jnp.concatenate(idxs, axis=-1)
    v = v / jnp.sum(v, axis=-1, keepdims=True)
    scores_ref[...] = v
    idx_ref[...] = i


def kernel(hidden_states, weight):
    hs = hidden_states.reshape(-1, _HIDDEN)
    n = hs.shape[0]
    grid = n // _ROWS_PER_BLOCK
    probs, scores, idx = pl.pallas_call(
        _router_kernel,
        grid=(grid,),
        in_specs=[
            pl.BlockSpec((_ROWS_PER_BLOCK, _HIDDEN), lambda i: (i, 0)),
            pl.BlockSpec((_NUM_EXPERTS, _HIDDEN), lambda i: (0, 0)),
        ],
        out_specs=[
            pl.BlockSpec((_ROWS_PER_BLOCK, _NUM_EXPERTS), lambda i: (i, 0)),
            pl.BlockSpec((_ROWS_PER_BLOCK, _TOP_K), lambda i: (i, 0)),
            pl.BlockSpec((_ROWS_PER_BLOCK, _TOP_K), lambda i: (i, 0)),
        ],
        out_shape=[
            jax.ShapeDtypeStruct((n, _NUM_EXPERTS), jnp.float32),
            jax.ShapeDtypeStruct((n, _TOP_K), jnp.float32),
            jax.ShapeDtypeStruct((n, _TOP_K), jnp.int32),
        ],
    )(hs, weight)
    return (probs, scores, idx)
